# baseline (speedup 1.0000x reference)
import dataclasses

import jax
import jax.numpy as jnp
from jax import lax
from jax.experimental import pallas as pl
from jax.experimental.pallas import tpu as pltpu
from jax.experimental.pallas import tpu_sc as plsc

_B, _N, _C, _S = 4, 16384, 64, 512
_TILE = 2048
_NT = _N // _TILE
_TW = 128


def _act_body(f_ref, xyz_ref, w1_ref, mean_ref, var_ref, gamma_ref, beta_ref,
              w2_ref, b2_ref, act_ref, z_ref, tab_ref):
    t = pl.program_id(1)
    f = f_ref[0]
    h = jnp.dot(w1_ref[...], f, preferred_element_type=jnp.float32)
    h = (h - mean_ref[...]) / jnp.sqrt(var_ref[...] + 1e-5) * gamma_ref[...] \
        + beta_ref[...]
    h = jnp.maximum(h, 0.0)
    lg = jnp.dot(w2_ref[...], h, preferred_element_type=jnp.float32)
    lg = lg + b2_ref[...]
    a = jnp.maximum(lg, 0.0) + jnp.log1p(jnp.exp(-jnp.abs(lg)))
    act_ref[0] = a

    @pl.when(t == 0)
    def _():
        z_ref[...] = jnp.zeros_like(z_ref)

    z_ref[...] += jnp.sum(a).reshape(1, 1, 1)

    tab_ref[0, :, 0:_C] = jnp.transpose(f)
    tab_ref[0, :, _C:_C + 3] = xyz_ref[0]


def _keys_body(act_ref, z_ref, g_ref, keys_ref):
    pw = act_ref[...] / (z_ref[...] + 1e-8)
    keys_ref[...] = jnp.log(pw + 1e-20) + g_ref[...]


def _compute_keys_and_table(points_xyz, features, W1, mean, var, gamma, beta,
                            W2, b2, gumbel):
    act, z, table = pl.pallas_call(
        _act_body,
        grid=(_B, _NT),
        in_specs=[
            pl.BlockSpec((1, _C, _TILE), lambda b, t: (b, 0, t)),
            pl.BlockSpec((1, _TILE, 3), lambda b, t: (b, t, 0)),
            pl.BlockSpec((_C, _C), lambda b, t: (0, 0)),
            pl.BlockSpec((_C, 1), lambda b, t: (0, 0)),
            pl.BlockSpec((_C, 1), lambda b, t: (0, 0)),
            pl.BlockSpec((_C, 1), lambda b, t: (0, 0)),
            pl.BlockSpec((_C, 1), lambda b, t: (0, 0)),
            pl.BlockSpec((1, _C), lambda b, t: (0, 0)),
            pl.BlockSpec((1, 1), lambda b, t: (0, 0)),
        ],
        out_specs=[
            pl.BlockSpec((1, 1, _TILE), lambda b, t: (b, 0, t)),
            pl.BlockSpec((1, 1, 1), lambda b, t: (b, 0, 0)),
            pl.BlockSpec((1, _TILE, _TW), lambda b, t: (b, t, 0)),
        ],
        out_shape=[
            jax.ShapeDtypeStruct((_B, 1, _N), jnp.float32),
            jax.ShapeDtypeStruct((_B, 1, 1), jnp.float32),
            jax.ShapeDtypeStruct((_B, _N, _TW), jnp.float32),
        ],
    )(features, points_xyz, W1, mean, var, gamma, beta, W2, b2)

    keys = pl.pallas_call(
        _keys_body,
        grid=(_B, _NT),
        in_specs=[
            pl.BlockSpec((1, 1, _TILE), lambda b, t: (b, 0, t)),
            pl.BlockSpec((1, 1, 1), lambda b, t: (b, 0, 0)),
            pl.BlockSpec((1, 1, _TILE), lambda b, t: (b, 0, t)),
        ],
        out_specs=pl.BlockSpec((1, 1, _TILE), lambda b, t: (b, 0, t)),
        out_shape=jax.ShapeDtypeStruct((_B, 1, _N), jnp.float32),
    )(act, z, gumbel)
    return keys[:, 0, :], table


_NSUB = 16
_WPB = 8
_SPW = _S // _WPB
_SH = _N // _WPB
_NCK = _SH // 16
_PAD0 = _S
_SOUT = _S + 16


def _monotone_i32(kf):
    b = lax.bitcast_convert_type(kf, jnp.int32)
    return b ^ ((b >> 31) & jnp.int32(0x7FFFFFFF))


def _sc_topk_gather_body(keys_hbm, tab_hbm, idx_hbm, orows_hbm,
                         keysb, mu, smu_loc, sidx_loc, cnt8, c88,
                         allmu, allidx, ranks_loc, vals_loc, idxg_v, rows_v,
                         cnts_spm, smu_spm, sidx_spm, sout_spm, sem):
    c = lax.axis_index("c")
    s = lax.axis_index("s")
    bl = s // _WPB
    b = c * 2 + bl
    g8 = s % _WPB

    ones = jnp.ones((16,), jnp.int32)
    zeros = jnp.zeros((16,), jnp.int32)
    iota16 = lax.iota(jnp.int32, 16)

    pltpu.sync_copy(keys_hbm.at[b, pl.ds(g8 * _SH, _SH)], keysb)

    @pl.loop(0, _NCK)
    def _(ch):
        mu[pl.ds(ch * 16, 16)] = _monotone_i32(keysb[pl.ds(ch * 16, 16)])

    def _count_ge(thr):
        def body(ch, acc):
            m = mu[pl.ds(ch * 16, 16)] >= thr
            return acc + jnp.where(m, ones, zeros)
        acc = lax.fori_loop(0, _NCK, body, zeros)
        return jnp.sum(acc)

    def _publish_and_sum(val):
        cnt8[...] = jnp.full((16,), val, jnp.int32)
        pltpu.sync_copy(cnt8, cnts_spm.at[pl.ds(bl * 256 + g8 * 32, 16)])
        plsc.subcore_barrier()
        pltpu.sync_copy(cnts_spm.at[pl.ds(bl * 256, 256)], c88)
        tot = jnp.int32(0)
        pre = jnp.int32(0)
        for j in range(_WPB):
            vj = c88[pl.ds(32 * j, 16)][0]
            tot = tot + vj
            pre = pre + jnp.where(jnp.int32(j) < g8, vj, 0)
        plsc.subcore_barrier()
        return tot, pre

    tot0, _ = _publish_and_sum(_count_ge(jnp.int32(0)))
    T = jnp.where(tot0 >= _S, jnp.int32(0), jnp.int32(-2**31))
    for bit in range(30, -1, -1):
        cand = T | jnp.int32(1 << bit)
        tot, _ = _publish_and_sum(_count_ge(cand))
        T = jnp.where(tot >= _S, cand, T)

    def cbody(ch, cnt):
        kv = mu[pl.ds(pl.multiple_of(ch * 16, 16), 16)]
        m = kv >= T
        mi = jnp.where(m, ones, zeros)
        pos = cnt + jnp.cumsum(mi) - 1
        plsc.store_scatter(smu_loc, [pos], kv, mask=m)
        gi = jnp.full((16,), g8 * _SH + ch * 16, jnp.int32) + iota16
        plsc.store_scatter(sidx_loc, [pos], gi, mask=m)
        return cnt + jnp.sum(mi)

    cntw = lax.fori_loop(0, _NCK, cbody, jnp.int32(0))
    padded = (cntw + 15) & ~jnp.int32(15)
    padm = iota16 < (padded - cntw)
    ppos = cntw + iota16
    plsc.store_scatter(smu_loc, [ppos],
                       jnp.full((16,), -2**31, jnp.int32), mask=padm)
    plsc.store_scatter(sidx_loc, [ppos], jnp.zeros((16,), jnp.int32),
                       mask=padm)

    totp, base = _publish_and_sum(padded)

    off = jnp.int32(0)
    for sz in (2048, 1024, 512, 256, 128, 64, 32, 16):
        cond = (padded & sz) != 0

        @pl.when(cond)
        def _():
            o1 = pl.multiple_of(off, 16)
            o2 = pl.multiple_of(bl * _N + base + off, 16)
            pltpu.sync_copy(smu_loc.at[pl.ds(o1, sz)],
                            smu_spm.at[pl.ds(o2, sz)])
            pltpu.sync_copy(sidx_loc.at[pl.ds(o1, sz)],
                            sidx_spm.at[pl.ds(o2, sz)])
        off = off + jnp.where(cond, sz, 0)
    plsc.subcore_barrier()

    off = jnp.int32(0)
    for sz in (16384, 8192, 4096, 2048, 1024, 512, 256, 128, 64, 32, 16):
        cond = (totp & sz) != 0

        @pl.when(cond)
        def _():
            o1 = pl.multiple_of(off, 16)
            o2 = pl.multiple_of(bl * _N + off, 16)
            pltpu.sync_copy(smu_spm.at[pl.ds(o2, sz)],
                            allmu.at[pl.ds(o1, sz)])
            pltpu.sync_copy(sidx_spm.at[pl.ds(o2, sz)],
                            allidx.at[pl.ds(o1, sz)])
        off = off + jnp.where(cond, sz, 0)

    @pl.loop(0, _SOUT // 16 + 1)
    def _(r):
        row = pl.multiple_of(2 * r, 2)
        ranks_loc[row] = jnp.full((16,), bl * _SOUT + _PAD0, jnp.int32) \
            + (iota16 & 7)

    nch = totp // 16

    def rbody(j, jout):
        jv = jnp.full((16,), j, jnp.int32)
        mu_jv = plsc.load_gather(smu_loc, [jv])
        idx_jv = plsc.load_gather(sidx_loc, [jv])

        def rin(ch, acc):
            cs = pl.multiple_of(ch * 16, 16)
            cmu = allmu[pl.ds(cs, 16)]
            cidx = allidx[pl.ds(cs, 16)]
            hit = (cmu > mu_jv) | ((cmu == mu_jv) & (cidx < idx_jv))
            return acc + jnp.where(hit, ones, zeros)

        rank = jnp.sum(lax.fori_loop(0, nch, rin, zeros))

        lane0 = iota16 == 0

        @pl.when(rank < _S)
        def _():
            r0 = jnp.full((16,), 2 * (jout // 16), jnp.int32)
            r1 = jnp.full((16,), jout % 16, jnp.int32)
            plsc.store_scatter(ranks_loc, [r0, r1],
                               jnp.full((16,), bl * _SOUT + rank, jnp.int32),
                               mask=lane0)
            plsc.store_scatter(vals_loc, [r0, r1], idx_jv, mask=lane0)

        return jout + jnp.where(rank < _S, 1, 0)

    jout = lax.fori_loop(0, cntw, rbody, jnp.int32(0))

    nout = (jout + 15) // 16

    def sbody(ch, _):
        row = pl.multiple_of(2 * ch, 2)
        pltpu.sync_copy(vals_loc.at[row],
                        sout_spm.at[ranks_loc.at[row]])
        return 0

    lax.fori_loop(0, nout, sbody, 0)
    plsc.subcore_barrier()

    pltpu.sync_copy(sout_spm.at[pl.ds(bl * _SOUT + g8 * _SPW, _SPW)], idxg_v)
    pltpu.sync_copy(idxg_v, idx_hbm.at[b, pl.ds(g8 * _SPW, _SPW)])
    pltpu.async_copy(tab_hbm.at[b].at[idxg_v], rows_v, sem).wait()
    pltpu.sync_copy(rows_v, orows_hbm.at[b, pl.ds(g8 * _SPW, _SPW)])


def _sc_compiler_params():
    cp = pltpu.CompilerParams()
    if "needs_layout_passes" in pltpu.CompilerParams.__dataclass_fields__:
        cp = dataclasses.replace(cp, needs_layout_passes=False)
    return cp


def _sc_topk_gather(keys, table):
    mesh = plsc.VectorSubcoreMesh(core_axis_name="c", subcore_axis_name="s")
    kfn = pl.kernel(
        _sc_topk_gather_body,
        mesh=mesh,
        compiler_params=_sc_compiler_params(),
        out_type=[
            jax.ShapeDtypeStruct((_B, _S), jnp.int32),
            jax.ShapeDtypeStruct((_B, _S, _TW), jnp.float32),
        ],
        scratch_types=[
            pltpu.VMEM((_SH,), jnp.float32),
            pltpu.VMEM((_SH,), jnp.int32),
            pltpu.VMEM((_SH + 16,), jnp.int32),
            pltpu.VMEM((_SH + 16,), jnp.int32),
            pltpu.VMEM((16,), jnp.int32),
            pltpu.VMEM((256,), jnp.int32),
            pltpu.VMEM((_N,), jnp.int32),
            pltpu.VMEM((_N,), jnp.int32),
            pltpu.VMEM((2 * (_SOUT // 16 + 1), 16), jnp.int32),
            pltpu.VMEM((2 * (_SOUT // 16 + 1), 16), jnp.int32),
            pltpu.VMEM((_SPW,), jnp.int32),
            pltpu.VMEM((_SPW, _TW), jnp.float32),
            pltpu.VMEM_SHARED((512,), jnp.int32),
            pltpu.VMEM_SHARED((2 * _N,), jnp.int32),
            pltpu.VMEM_SHARED((2 * _N,), jnp.int32),
            pltpu.VMEM_SHARED((2 * _SOUT,), jnp.int32),
            pltpu.SemaphoreType.DMA,
        ],
    )
    return kfn(keys, table)


def _untangle_body(rows_ref, oxyz_ref, ofea_ref):
    g = rows_ref[0]
    ofea_ref[0] = jnp.transpose(g[:, 0:_C])
    oxyz_ref[0] = g[:, _C:_C + 3]


def _untangle(rows):
    return pl.pallas_call(
        _untangle_body,
        grid=(_B,),
        in_specs=[pl.BlockSpec((1, _S, _TW), lambda b: (b, 0, 0))],
        out_specs=[
            pl.BlockSpec((1, _S, 3), lambda b: (b, 0, 0)),
            pl.BlockSpec((1, _C, _S), lambda b: (b, 0, 0)),
        ],
        out_shape=[
            jax.ShapeDtypeStruct((_B, _S, 3), jnp.float32),
            jax.ShapeDtypeStruct((_B, _C, _S), jnp.float32),
        ],
    )(rows)


def kernel(points_xyz, features, W1, gamma, beta, running_mean, running_var,
           W2, b2):
    u = jax.random.uniform(jax.random.key(42), (_B, _N),
                           minval=1e-10, maxval=1.0)
    gumbel = -jnp.log(-jnp.log(u))

    keys, table = _compute_keys_and_table(
        points_xyz, features, W1, running_mean[:, None], running_var[:, None],
        gamma[:, None], beta[:, None], W2, b2[:, None], gumbel[:, None, :])
    idx, rows = _sc_topk_gather(keys, table)
    new_xyz, new_fea = _untangle(rows)
    return new_xyz, new_fea, idx


# speedup vs baseline: 1.0771x; 1.0771x over previous
import dataclasses

import jax
import jax.numpy as jnp
from jax import lax
from jax.experimental import pallas as pl
from jax.experimental.pallas import tpu as pltpu
from jax.experimental.pallas import tpu_sc as plsc

_B, _N, _C, _S = 4, 16384, 64, 512
_TILE = 2048
_NT = _N // _TILE
_TW = 128


def _act_body(f_ref, xyz_ref, w1_ref, mean_ref, var_ref, gamma_ref, beta_ref,
              w2_ref, b2_ref, act_ref, z_ref, tab_ref):
    t = pl.program_id(1)
    f = f_ref[0]
    h = jnp.dot(w1_ref[...], f, preferred_element_type=jnp.float32)
    h = (h - mean_ref[...]) / jnp.sqrt(var_ref[...] + 1e-5) * gamma_ref[...] \
        + beta_ref[...]
    h = jnp.maximum(h, 0.0)
    lg = jnp.dot(w2_ref[...], h, preferred_element_type=jnp.float32)
    lg = lg + b2_ref[...]
    a = jnp.maximum(lg, 0.0) + jnp.log1p(jnp.exp(-jnp.abs(lg)))
    act_ref[0] = a

    @pl.when(t == 0)
    def _():
        z_ref[...] = jnp.zeros_like(z_ref)

    z_ref[...] += jnp.sum(a).reshape(1, 1, 1)

    fT = jnp.transpose(f)
    pad = jnp.zeros((_TILE, _TW - _C - 3), jnp.float32)
    tab_ref[0] = jnp.concatenate([fT, xyz_ref[0], pad], axis=1)


def _keys_body(act_ref, z_ref, g_ref, keys_ref):
    pw = act_ref[...] / (z_ref[...] + 1e-8)
    keys_ref[...] = jnp.log(pw + 1e-20) + g_ref[...]


def _compute_keys_and_table(points_xyz, features, W1, mean, var, gamma, beta,
                            W2, b2, gumbel):
    act, z, table = pl.pallas_call(
        _act_body,
        grid=(_B, _NT),
        in_specs=[
            pl.BlockSpec((1, _C, _TILE), lambda b, t: (b, 0, t)),
            pl.BlockSpec((1, _TILE, 3), lambda b, t: (b, t, 0)),
            pl.BlockSpec((_C, _C), lambda b, t: (0, 0)),
            pl.BlockSpec((_C, 1), lambda b, t: (0, 0)),
            pl.BlockSpec((_C, 1), lambda b, t: (0, 0)),
            pl.BlockSpec((_C, 1), lambda b, t: (0, 0)),
            pl.BlockSpec((_C, 1), lambda b, t: (0, 0)),
            pl.BlockSpec((1, _C), lambda b, t: (0, 0)),
            pl.BlockSpec((1, 1), lambda b, t: (0, 0)),
        ],
        out_specs=[
            pl.BlockSpec((1, 1, _TILE), lambda b, t: (b, 0, t)),
            pl.BlockSpec((1, 1, 1), lambda b, t: (b, 0, 0)),
            pl.BlockSpec((1, _TILE, _TW), lambda b, t: (b, t, 0)),
        ],
        out_shape=[
            jax.ShapeDtypeStruct((_B, 1, _N), jnp.float32),
            jax.ShapeDtypeStruct((_B, 1, 1), jnp.float32),
            jax.ShapeDtypeStruct((_B, _N, _TW), jnp.float32),
        ],
    )(features, points_xyz, W1, mean, var, gamma, beta, W2, b2)

    keys = pl.pallas_call(
        _keys_body,
        grid=(_B, _NT),
        in_specs=[
            pl.BlockSpec((1, 1, _TILE), lambda b, t: (b, 0, t)),
            pl.BlockSpec((1, 1, 1), lambda b, t: (b, 0, 0)),
            pl.BlockSpec((1, 1, _TILE), lambda b, t: (b, 0, t)),
        ],
        out_specs=pl.BlockSpec((1, 1, _TILE), lambda b, t: (b, 0, t)),
        out_shape=jax.ShapeDtypeStruct((_B, 1, _N), jnp.float32),
    )(act, z, gumbel)
    return keys[:, 0, :], table


_NSUB = 16
_WPB = 8
_SPW = _S // _WPB
_SH = _N // _WPB
_NCK = _SH // 16
_PAD0 = _S
_SOUT = _S + 16


def _monotone_i32(kf):
    b = lax.bitcast_convert_type(kf, jnp.int32)
    return b ^ ((b >> 31) & jnp.int32(0x7FFFFFFF))


def _sc_topk_gather_body(keys_hbm, tab_hbm, idx_hbm, orows_hbm,
                         keysb, mu, smu_loc, sidx_loc, cnt8, c88,
                         allmu, allidx, ranks_loc, vals_loc, idxg_v, rows_v,
                         cnts_spm, smu_spm, sidx_spm, sout_spm, sem):
    c = lax.axis_index("c")
    s = lax.axis_index("s")
    bl = s // _WPB
    b = c * 2 + bl
    g8 = s % _WPB

    ones = jnp.ones((16,), jnp.int32)
    zeros = jnp.zeros((16,), jnp.int32)
    iota16 = lax.iota(jnp.int32, 16)

    pltpu.sync_copy(keys_hbm.at[b, pl.ds(g8 * _SH, _SH)], keysb)

    @pl.loop(0, _NCK)
    def _(ch):
        mu[pl.ds(ch * 16, 16)] = _monotone_i32(keysb[pl.ds(ch * 16, 16)])

    def _count_ge(thr):
        def body(ch, acc):
            base = pl.multiple_of(ch * 64, 64)
            for u in range(4):
                m = mu[pl.ds(base + u * 16, 16)] >= thr
                acc = acc + jnp.where(m, ones, zeros)
            return acc
        acc = lax.fori_loop(0, _NCK // 4, body, zeros)
        return jnp.sum(acc)

    def _publish_and_sum(val, slot):
        cnt8[...] = jnp.full((16,), val, jnp.int32)
        pltpu.sync_copy(
            cnt8, cnts_spm.at[pl.ds(slot * 512 + bl * 256 + g8 * 32, 16)])
        plsc.subcore_barrier()
        pltpu.sync_copy(cnts_spm.at[pl.ds(slot * 512 + bl * 256, 256)], c88)
        tot = jnp.int32(0)
        pre = jnp.int32(0)
        for j in range(_WPB):
            vj = c88[pl.ds(32 * j, 16)][0]
            tot = tot + vj
            pre = pre + jnp.where(jnp.int32(j) < g8, vj, 0)
        return tot, pre

    tot0, _ = _publish_and_sum(_count_ge(jnp.int32(0)), 0)
    T = jnp.where(tot0 >= _S, jnp.int32(0), jnp.int32(-2**31))
    for bit in range(30, -1, -1):
        cand = T | jnp.int32(1 << bit)
        tot, _ = _publish_and_sum(_count_ge(cand), (31 - bit) % 2)
        T = jnp.where(tot >= _S, cand, T)

    def cbody(ch, cnt):
        kv = mu[pl.ds(pl.multiple_of(ch * 16, 16), 16)]
        m = kv >= T
        mi = jnp.where(m, ones, zeros)
        pos = cnt + jnp.cumsum(mi) - 1
        plsc.store_scatter(smu_loc, [pos], kv, mask=m)
        gi = jnp.full((16,), g8 * _SH + ch * 16, jnp.int32) + iota16
        plsc.store_scatter(sidx_loc, [pos], gi, mask=m)
        return cnt + jnp.sum(mi)

    cntw = lax.fori_loop(0, _NCK, cbody, jnp.int32(0))
    padded = (cntw + 15) & ~jnp.int32(15)
    padm = iota16 < (padded - cntw)
    ppos = cntw + iota16
    plsc.store_scatter(smu_loc, [ppos],
                       jnp.full((16,), -2**31, jnp.int32), mask=padm)
    plsc.store_scatter(sidx_loc, [ppos], jnp.zeros((16,), jnp.int32),
                       mask=padm)

    totp, base = _publish_and_sum(padded, 0)

    off = jnp.int32(0)
    for sz in (2048, 1024, 512, 256, 128, 64, 32, 16):
        cond = (padded & sz) != 0

        @pl.when(cond)
        def _():
            o1 = pl.multiple_of(off, 16)
            o2 = pl.multiple_of(bl * _N + base + off, 16)
            pltpu.sync_copy(smu_loc.at[pl.ds(o1, sz)],
                            smu_spm.at[pl.ds(o2, sz)])
            pltpu.sync_copy(sidx_loc.at[pl.ds(o1, sz)],
                            sidx_spm.at[pl.ds(o2, sz)])
        off = off + jnp.where(cond, sz, 0)
    plsc.subcore_barrier()

    off = jnp.int32(0)
    for sz in (16384, 8192, 4096, 2048, 1024, 512, 256, 128, 64, 32, 16):
        cond = (totp & sz) != 0

        @pl.when(cond)
        def _():
            o1 = pl.multiple_of(off, 16)
            o2 = pl.multiple_of(bl * _N + off, 16)
            pltpu.sync_copy(smu_spm.at[pl.ds(o2, sz)],
                            allmu.at[pl.ds(o1, sz)])
            pltpu.sync_copy(sidx_spm.at[pl.ds(o2, sz)],
                            allidx.at[pl.ds(o1, sz)])
        off = off + jnp.where(cond, sz, 0)

    @pl.loop(0, _SOUT // 16 + 1)
    def _(r):
        row = pl.multiple_of(2 * r, 2)
        ranks_loc[row] = jnp.full((16,), bl * _SOUT + _PAD0, jnp.int32) \
            + (iota16 & 7)

    nch = totp // 16

    def rbody(j, jout):
        jv = jnp.full((16,), j, jnp.int32)
        mu_jv = plsc.load_gather(smu_loc, [jv])
        idx_jv = plsc.load_gather(sidx_loc, [jv])

        def rin(ch, acc):
            cs = pl.multiple_of(ch * 16, 16)
            cmu = allmu[pl.ds(cs, 16)]
            cidx = allidx[pl.ds(cs, 16)]
            hit = (cmu > mu_jv) | ((cmu == mu_jv) & (cidx < idx_jv))
            return acc + jnp.where(hit, ones, zeros)

        rank = jnp.sum(lax.fori_loop(0, nch, rin, zeros))

        lane0 = iota16 == 0

        @pl.when(rank < _S)
        def _():
            r0 = jnp.full((16,), 2 * (jout // 16), jnp.int32)
            r1 = jnp.full((16,), jout % 16, jnp.int32)
            plsc.store_scatter(ranks_loc, [r0, r1],
                               jnp.full((16,), bl * _SOUT + rank, jnp.int32),
                               mask=lane0)
            plsc.store_scatter(vals_loc, [r0, r1], idx_jv, mask=lane0)

        return jout + jnp.where(rank < _S, 1, 0)

    jout = lax.fori_loop(0, cntw, rbody, jnp.int32(0))

    nout = (jout + 15) // 16

    def sbody(ch, _):
        row = pl.multiple_of(2 * ch, 2)
        pltpu.sync_copy(vals_loc.at[row],
                        sout_spm.at[ranks_loc.at[row]])
        return 0

    lax.fori_loop(0, nout, sbody, 0)
    plsc.subcore_barrier()

    pltpu.sync_copy(sout_spm.at[pl.ds(bl * _SOUT + g8 * _SPW, _SPW)], idxg_v)
    pltpu.sync_copy(idxg_v, idx_hbm.at[b, pl.ds(g8 * _SPW, _SPW)])
    pltpu.async_copy(tab_hbm.at[b].at[idxg_v], rows_v, sem).wait()
    pltpu.sync_copy(rows_v, orows_hbm.at[b, pl.ds(g8 * _SPW, _SPW)])


def _sc_compiler_params():
    cp = pltpu.CompilerParams()
    if "needs_layout_passes" in pltpu.CompilerParams.__dataclass_fields__:
        cp = dataclasses.replace(cp, needs_layout_passes=False)
    return cp


def _sc_topk_gather(keys, table):
    mesh = plsc.VectorSubcoreMesh(core_axis_name="c", subcore_axis_name="s")
    kfn = pl.kernel(
        _sc_topk_gather_body,
        mesh=mesh,
        compiler_params=_sc_compiler_params(),
        out_type=[
            jax.ShapeDtypeStruct((_B, _S), jnp.int32),
            jax.ShapeDtypeStruct((_B, _S, _TW), jnp.float32),
        ],
        scratch_types=[
            pltpu.VMEM((_SH,), jnp.float32),
            pltpu.VMEM((_SH,), jnp.int32),
            pltpu.VMEM((_SH + 16,), jnp.int32),
            pltpu.VMEM((_SH + 16,), jnp.int32),
            pltpu.VMEM((16,), jnp.int32),
            pltpu.VMEM((256,), jnp.int32),
            pltpu.VMEM((_N,), jnp.int32),
            pltpu.VMEM((_N,), jnp.int32),
            pltpu.VMEM((2 * (_SOUT // 16 + 1), 16), jnp.int32),
            pltpu.VMEM((2 * (_SOUT // 16 + 1), 16), jnp.int32),
            pltpu.VMEM((_SPW,), jnp.int32),
            pltpu.VMEM((_SPW, _TW), jnp.float32),
            pltpu.VMEM_SHARED((1024,), jnp.int32),
            pltpu.VMEM_SHARED((2 * _N,), jnp.int32),
            pltpu.VMEM_SHARED((2 * _N,), jnp.int32),
            pltpu.VMEM_SHARED((2 * _SOUT,), jnp.int32),
            pltpu.SemaphoreType.DMA,
        ],
    )
    return kfn(keys, table)


def _untangle_body(rows_ref, oxyz_ref, ofea_ref):
    g = rows_ref[0]
    ofea_ref[0] = jnp.transpose(g[:, 0:_C])
    oxyz_ref[0] = g[:, _C:_C + 3]


def _untangle(rows):
    return pl.pallas_call(
        _untangle_body,
        grid=(_B,),
        in_specs=[pl.BlockSpec((1, _S, _TW), lambda b: (b, 0, 0))],
        out_specs=[
            pl.BlockSpec((1, _S, 3), lambda b: (b, 0, 0)),
            pl.BlockSpec((1, _C, _S), lambda b: (b, 0, 0)),
        ],
        out_shape=[
            jax.ShapeDtypeStruct((_B, _S, 3), jnp.float32),
            jax.ShapeDtypeStruct((_B, _C, _S), jnp.float32),
        ],
    )(rows)


def kernel(points_xyz, features, W1, gamma, beta, running_mean, running_var,
           W2, b2):
    u = jax.random.uniform(jax.random.key(42), (_B, _N),
                           minval=1e-10, maxval=1.0)
    gumbel = -jnp.log(-jnp.log(u))

    keys, table = _compute_keys_and_table(
        points_xyz, features, W1, running_mean[:, None], running_var[:, None],
        gamma[:, None], beta[:, None], W2, b2[:, None], gumbel[:, None, :])
    idx, rows = _sc_topk_gather(keys, table)
    new_xyz, new_fea = _untangle(rows)
    return new_xyz, new_fea, idx


# speedup vs baseline: 1.1159x; 1.0360x over previous
import dataclasses

import jax
import jax.numpy as jnp
from jax import lax
from jax.experimental import pallas as pl
from jax.experimental.pallas import tpu as pltpu
from jax.experimental.pallas import tpu_sc as plsc

_B, _N, _C, _S = 4, 16384, 64, 512
_TILE = 2048
_NT = _N // _TILE
_TW = 128


def _act_body(f_ref, w1_ref, mean_ref, var_ref, gamma_ref, beta_ref,
              w2_ref, b2_ref, act_ref, z_ref):
    t = pl.program_id(1)
    f = f_ref[0]
    h = jnp.dot(w1_ref[...], f, preferred_element_type=jnp.float32)
    h = (h - mean_ref[...]) / jnp.sqrt(var_ref[...] + 1e-5) * gamma_ref[...] \
        + beta_ref[...]
    h = jnp.maximum(h, 0.0)
    lg = jnp.dot(w2_ref[...], h, preferred_element_type=jnp.float32)
    lg = lg + b2_ref[...]
    a = jnp.maximum(lg, 0.0) + jnp.log1p(jnp.exp(-jnp.abs(lg)))
    act_ref[0] = a

    @pl.when(t == 0)
    def _():
        z_ref[...] = jnp.zeros_like(z_ref)

    z_ref[...] += jnp.sum(a).reshape(1, 1, 1)


def _table_body(f_ref, xyz_ref, tab_ref):
    fT = jnp.transpose(f_ref[0])
    pad = jnp.zeros((_TILE, _TW - _C - 3), jnp.float32)
    tab_ref[0] = jnp.concatenate([fT, xyz_ref[0], pad], axis=1)


def _keys_body(act_ref, z_ref, g_ref, keys_ref):
    pw = act_ref[...] / (z_ref[...] + 1e-8)
    keys_ref[...] = jnp.log(pw + 1e-20) + g_ref[...]


def _build_table(points_xyz, features):
    return pl.pallas_call(
        _table_body,
        grid=(_B, _NT),
        in_specs=[
            pl.BlockSpec((1, _C, _TILE), lambda b, t: (b, 0, t)),
            pl.BlockSpec((1, _TILE, 3), lambda b, t: (b, t, 0)),
        ],
        out_specs=pl.BlockSpec((1, _TILE, _TW), lambda b, t: (b, t, 0)),
        out_shape=jax.ShapeDtypeStruct((_B, _N, _TW), jnp.float32),
    )(features, points_xyz)


def _compute_keys(features, W1, mean, var, gamma, beta, W2, b2, gumbel):
    act, z = pl.pallas_call(
        _act_body,
        grid=(_B, _NT),
        in_specs=[
            pl.BlockSpec((1, _C, _TILE), lambda b, t: (b, 0, t)),
            pl.BlockSpec((_C, _C), lambda b, t: (0, 0)),
            pl.BlockSpec((_C, 1), lambda b, t: (0, 0)),
            pl.BlockSpec((_C, 1), lambda b, t: (0, 0)),
            pl.BlockSpec((_C, 1), lambda b, t: (0, 0)),
            pl.BlockSpec((_C, 1), lambda b, t: (0, 0)),
            pl.BlockSpec((1, _C), lambda b, t: (0, 0)),
            pl.BlockSpec((1, 1), lambda b, t: (0, 0)),
        ],
        out_specs=[
            pl.BlockSpec((1, 1, _TILE), lambda b, t: (b, 0, t)),
            pl.BlockSpec((1, 1, 1), lambda b, t: (b, 0, 0)),
        ],
        out_shape=[
            jax.ShapeDtypeStruct((_B, 1, _N), jnp.float32),
            jax.ShapeDtypeStruct((_B, 1, 1), jnp.float32),
        ],
    )(features, W1, mean, var, gamma, beta, W2, b2)

    keys = pl.pallas_call(
        _keys_body,
        grid=(_B, _NT),
        in_specs=[
            pl.BlockSpec((1, 1, _TILE), lambda b, t: (b, 0, t)),
            pl.BlockSpec((1, 1, 1), lambda b, t: (b, 0, 0)),
            pl.BlockSpec((1, 1, _TILE), lambda b, t: (b, 0, t)),
        ],
        out_specs=pl.BlockSpec((1, 1, _TILE), lambda b, t: (b, 0, t)),
        out_shape=jax.ShapeDtypeStruct((_B, 1, _N), jnp.float32),
    )(act, z, gumbel)
    return keys[:, 0, :]


_NSUB = 16
_WPB = 8
_SPW = _S // _WPB
_SH = _N // _WPB
_NCK = _SH // 16
_PAD0 = _S
_SOUT = _S + 16


def _monotone_i32(kf):
    b = lax.bitcast_convert_type(kf, jnp.int32)
    return b ^ ((b >> 31) & jnp.int32(0x7FFFFFFF))


def _sc_topk_body(keys_hbm, idx_hbm,
                  keysb, mu, smu_loc, sidx_loc, cnt8, c88,
                  allmu, allidx, ranks_loc, vals_loc, idxg_v,
                  cnts_spm, smu_spm, sidx_spm, sout_spm):
    c = lax.axis_index("c")
    s = lax.axis_index("s")
    bl = s // _WPB
    b = c * 2 + bl
    g8 = s % _WPB

    ones = jnp.ones((16,), jnp.int32)
    zeros = jnp.zeros((16,), jnp.int32)
    iota16 = lax.iota(jnp.int32, 16)

    pltpu.sync_copy(keys_hbm.at[b, pl.ds(g8 * _SH, _SH)], keysb)

    @pl.loop(0, _NCK)
    def _(ch):
        mu[pl.ds(ch * 16, 16)] = _monotone_i32(keysb[pl.ds(ch * 16, 16)])

    def _count_ge(thr):
        def body(ch, acc):
            base = pl.multiple_of(ch * 64, 64)
            for u in range(4):
                m = mu[pl.ds(base + u * 16, 16)] >= thr
                acc = acc + jnp.where(m, ones, zeros)
            return acc
        acc = lax.fori_loop(0, _NCK // 4, body, zeros)
        return jnp.sum(acc)

    def _publish_and_sum(val, slot):
        cnt8[...] = jnp.full((16,), val, jnp.int32)
        pltpu.sync_copy(
            cnt8, cnts_spm.at[pl.ds(slot * 512 + bl * 256 + g8 * 32, 16)])
        plsc.subcore_barrier()
        pltpu.sync_copy(cnts_spm.at[pl.ds(slot * 512 + bl * 256, 256)], c88)
        tot = jnp.int32(0)
        pre = jnp.int32(0)
        for j in range(_WPB):
            vj = c88[pl.ds(32 * j, 16)][0]
            tot = tot + vj
            pre = pre + jnp.where(jnp.int32(j) < g8, vj, 0)
        return tot, pre

    tot0, _ = _publish_and_sum(_count_ge(jnp.int32(0)), 0)
    T = jnp.where(tot0 >= _S, jnp.int32(0), jnp.int32(-2**31))
    for bit in range(30, -1, -1):
        cand = T | jnp.int32(1 << bit)
        tot, _ = _publish_and_sum(_count_ge(cand), (31 - bit) % 2)
        T = jnp.where(tot >= _S, cand, T)

    def cbody(ch, cnt):
        kv = mu[pl.ds(pl.multiple_of(ch * 16, 16), 16)]
        m = kv >= T
        mi = jnp.where(m, ones, zeros)
        pos = cnt + jnp.cumsum(mi) - 1
        plsc.store_scatter(smu_loc, [pos], kv, mask=m)
        gi = jnp.full((16,), g8 * _SH + ch * 16, jnp.int32) + iota16
        plsc.store_scatter(sidx_loc, [pos], gi, mask=m)
        return cnt + jnp.sum(mi)

    cntw = lax.fori_loop(0, _NCK, cbody, jnp.int32(0))
    padded = (cntw + 15) & ~jnp.int32(15)
    padm = iota16 < (padded - cntw)
    ppos = cntw + iota16
    plsc.store_scatter(smu_loc, [ppos],
                       jnp.full((16,), -2**31, jnp.int32), mask=padm)
    plsc.store_scatter(sidx_loc, [ppos], jnp.zeros((16,), jnp.int32),
                       mask=padm)

    totp, base = _publish_and_sum(padded, 0)

    off = jnp.int32(0)
    for sz in (2048, 1024, 512, 256, 128, 64, 32, 16):
        cond = (padded & sz) != 0

        @pl.when(cond)
        def _():
            o1 = pl.multiple_of(off, 16)
            o2 = pl.multiple_of(bl * _N + base + off, 16)
            pltpu.sync_copy(smu_loc.at[pl.ds(o1, sz)],
                            smu_spm.at[pl.ds(o2, sz)])
            pltpu.sync_copy(sidx_loc.at[pl.ds(o1, sz)],
                            sidx_spm.at[pl.ds(o2, sz)])
        off = off + jnp.where(cond, sz, 0)
    plsc.subcore_barrier()

    off = jnp.int32(0)
    for sz in (16384, 8192, 4096, 2048, 1024, 512, 256, 128, 64, 32, 16):
        cond = (totp & sz) != 0

        @pl.when(cond)
        def _():
            o1 = pl.multiple_of(off, 16)
            o2 = pl.multiple_of(bl * _N + off, 16)
            pltpu.sync_copy(smu_spm.at[pl.ds(o2, sz)],
                            allmu.at[pl.ds(o1, sz)])
            pltpu.sync_copy(sidx_spm.at[pl.ds(o2, sz)],
                            allidx.at[pl.ds(o1, sz)])
        off = off + jnp.where(cond, sz, 0)

    @pl.loop(0, _SOUT // 16 + 1)
    def _(r):
        row = pl.multiple_of(2 * r, 2)
        ranks_loc[row] = jnp.full((16,), bl * _SOUT + _PAD0, jnp.int32) \
            + (iota16 & 7)

    nch = totp // 16

    def rbody(j, jout):
        jv = jnp.full((16,), j, jnp.int32)
        mu_jv = plsc.load_gather(smu_loc, [jv])
        idx_jv = plsc.load_gather(sidx_loc, [jv])

        def rin(ch, acc):
            cs = pl.multiple_of(ch * 16, 16)
            cmu = allmu[pl.ds(cs, 16)]
            cidx = allidx[pl.ds(cs, 16)]
            hit = (cmu > mu_jv) | ((cmu == mu_jv) & (cidx < idx_jv))
            return acc + jnp.where(hit, ones, zeros)

        rank = jnp.sum(lax.fori_loop(0, nch, rin, zeros))

        lane0 = iota16 == 0

        @pl.when(rank < _S)
        def _():
            r0 = jnp.full((16,), 2 * (jout // 16), jnp.int32)
            r1 = jnp.full((16,), jout % 16, jnp.int32)
            plsc.store_scatter(ranks_loc, [r0, r1],
                               jnp.full((16,), bl * _SOUT + rank, jnp.int32),
                               mask=lane0)
            plsc.store_scatter(vals_loc, [r0, r1], idx_jv, mask=lane0)

        return jout + jnp.where(rank < _S, 1, 0)

    jout = lax.fori_loop(0, cntw, rbody, jnp.int32(0))

    nout = (jout + 15) // 16

    def sbody(ch, _):
        row = pl.multiple_of(2 * ch, 2)
        pltpu.sync_copy(vals_loc.at[row],
                        sout_spm.at[ranks_loc.at[row]])
        return 0

    lax.fori_loop(0, nout, sbody, 0)
    plsc.subcore_barrier()

    pltpu.sync_copy(sout_spm.at[pl.ds(bl * _SOUT + g8 * _SPW, _SPW)], idxg_v)
    pltpu.sync_copy(idxg_v, idx_hbm.at[b, pl.ds(g8 * _SPW, _SPW)])


def _sc_gather_body(idx_hbm, tab_hbm, orows_hbm, idxg_v, rows_v, sem):
    c = lax.axis_index("c")
    s = lax.axis_index("s")
    w = c * _NSUB + s
    b = w // _WPB
    g8 = w % _WPB
    pltpu.sync_copy(idx_hbm.at[b, pl.ds(g8 * _SPW, _SPW)], idxg_v)
    pltpu.async_copy(tab_hbm.at[b].at[idxg_v], rows_v, sem).wait()
    pltpu.sync_copy(rows_v, orows_hbm.at[b, pl.ds(g8 * _SPW, _SPW)])


def _sc_compiler_params():
    cp = pltpu.CompilerParams()
    if "needs_layout_passes" in pltpu.CompilerParams.__dataclass_fields__:
        cp = dataclasses.replace(cp, needs_layout_passes=False)
    return cp


def _sc_topk(keys):
    mesh = plsc.VectorSubcoreMesh(core_axis_name="c", subcore_axis_name="s")
    kfn = pl.kernel(
        _sc_topk_body,
        mesh=mesh,
        compiler_params=_sc_compiler_params(),
        out_type=jax.ShapeDtypeStruct((_B, _S), jnp.int32),
        scratch_types=[
            pltpu.VMEM((_SH,), jnp.float32),
            pltpu.VMEM((_SH,), jnp.int32),
            pltpu.VMEM((_SH + 16,), jnp.int32),
            pltpu.VMEM((_SH + 16,), jnp.int32),
            pltpu.VMEM((16,), jnp.int32),
            pltpu.VMEM((256,), jnp.int32),
            pltpu.VMEM((_N,), jnp.int32),
            pltpu.VMEM((_N,), jnp.int32),
            pltpu.VMEM((2 * (_SOUT // 16 + 1), 16), jnp.int32),
            pltpu.VMEM((2 * (_SOUT // 16 + 1), 16), jnp.int32),
            pltpu.VMEM((_SPW,), jnp.int32),
            pltpu.VMEM_SHARED((1024,), jnp.int32),
            pltpu.VMEM_SHARED((2 * _N,), jnp.int32),
            pltpu.VMEM_SHARED((2 * _N,), jnp.int32),
            pltpu.VMEM_SHARED((2 * _SOUT,), jnp.int32),
        ],
    )
    return kfn(keys)


def _sc_gather(idx, table):
    mesh = plsc.VectorSubcoreMesh(core_axis_name="c", subcore_axis_name="s")
    kfn = pl.kernel(
        _sc_gather_body,
        mesh=mesh,
        compiler_params=_sc_compiler_params(),
        out_type=jax.ShapeDtypeStruct((_B, _S, _TW), jnp.float32),
        scratch_types=[
            pltpu.VMEM((_SPW,), jnp.int32),
            pltpu.VMEM((_SPW, _TW), jnp.float32),
            pltpu.SemaphoreType.DMA,
        ],
    )
    return kfn(idx, table)


def _untangle_body(rows_ref, oxyz_ref, ofea_ref):
    g = rows_ref[0]
    ofea_ref[0] = jnp.transpose(g[:, 0:_C])
    oxyz_ref[0] = g[:, _C:_C + 3]


def _untangle(rows):
    return pl.pallas_call(
        _untangle_body,
        grid=(_B,),
        in_specs=[pl.BlockSpec((1, _S, _TW), lambda b: (b, 0, 0))],
        out_specs=[
            pl.BlockSpec((1, _S, 3), lambda b: (b, 0, 0)),
            pl.BlockSpec((1, _C, _S), lambda b: (b, 0, 0)),
        ],
        out_shape=[
            jax.ShapeDtypeStruct((_B, _S, 3), jnp.float32),
            jax.ShapeDtypeStruct((_B, _C, _S), jnp.float32),
        ],
    )(rows)


def kernel(points_xyz, features, W1, gamma, beta, running_mean, running_var,
           W2, b2):
    u = jax.random.uniform(jax.random.key(42), (_B, _N),
                           minval=1e-10, maxval=1.0)
    gumbel = -jnp.log(-jnp.log(u))

    keys = _compute_keys(
        features, W1, running_mean[:, None], running_var[:, None],
        gamma[:, None], beta[:, None], W2, b2[:, None], gumbel[:, None, :])
    table = _build_table(points_xyz, features)
    idx = _sc_topk(keys)
    rows = _sc_gather(idx, table)
    new_xyz, new_fea = _untangle(rows)
    return new_xyz, new_fea, idx


# speedup vs baseline: 1.1161x; 1.0002x over previous
import dataclasses

import jax
import jax.numpy as jnp
from jax import lax
from jax.experimental import pallas as pl
from jax.experimental.pallas import tpu as pltpu
from jax.experimental.pallas import tpu_sc as plsc

_B, _N, _C, _S = 4, 16384, 64, 512
_TILE = 2048
_NT = _N // _TILE
_TW = 128


def _act_body(f_ref, w1_ref, mean_ref, var_ref, gamma_ref, beta_ref,
              w2_ref, b2_ref, act_ref, z_ref):
    t = pl.program_id(1)
    f = f_ref[0]
    h = jnp.dot(w1_ref[...], f, preferred_element_type=jnp.float32)
    h = (h - mean_ref[...]) / jnp.sqrt(var_ref[...] + 1e-5) * gamma_ref[...] \
        + beta_ref[...]
    h = jnp.maximum(h, 0.0)
    lg = jnp.dot(w2_ref[...], h, preferred_element_type=jnp.float32)
    lg = lg + b2_ref[...]
    a = jnp.maximum(lg, 0.0) + jnp.log1p(jnp.exp(-jnp.abs(lg)))
    act_ref[0] = a

    @pl.when(t == 0)
    def _():
        z_ref[...] = jnp.zeros_like(z_ref)

    z_ref[...] += jnp.sum(a).reshape(1, 1, 1)


def _table_body(f_ref, xyz_ref, tab_ref):
    fT = jnp.transpose(f_ref[0])
    pad = jnp.zeros((_TILE, _TW - _C - 3), jnp.float32)
    tab_ref[0] = jnp.concatenate([fT, xyz_ref[0], pad], axis=1)


def _keys_body(act_ref, z_ref, g_ref, keys_ref):
    pw = act_ref[...] / (z_ref[...] + 1e-8)
    keys_ref[...] = jnp.log(pw + 1e-20) + g_ref[...]


def _build_table(points_xyz, features):
    return pl.pallas_call(
        _table_body,
        grid=(_B, _NT),
        in_specs=[
            pl.BlockSpec((1, _C, _TILE), lambda b, t: (b, 0, t)),
            pl.BlockSpec((1, _TILE, 3), lambda b, t: (b, t, 0)),
        ],
        out_specs=pl.BlockSpec((1, _TILE, _TW), lambda b, t: (b, t, 0)),
        out_shape=jax.ShapeDtypeStruct((_B, _N, _TW), jnp.float32),
    )(features, points_xyz)


def _compute_keys(features, W1, mean, var, gamma, beta, W2, b2, gumbel):
    act, z = pl.pallas_call(
        _act_body,
        grid=(_B, _NT),
        in_specs=[
            pl.BlockSpec((1, _C, _TILE), lambda b, t: (b, 0, t)),
            pl.BlockSpec((_C, _C), lambda b, t: (0, 0)),
            pl.BlockSpec((_C, 1), lambda b, t: (0, 0)),
            pl.BlockSpec((_C, 1), lambda b, t: (0, 0)),
            pl.BlockSpec((_C, 1), lambda b, t: (0, 0)),
            pl.BlockSpec((_C, 1), lambda b, t: (0, 0)),
            pl.BlockSpec((1, _C), lambda b, t: (0, 0)),
            pl.BlockSpec((1, 1), lambda b, t: (0, 0)),
        ],
        out_specs=[
            pl.BlockSpec((1, 1, _TILE), lambda b, t: (b, 0, t)),
            pl.BlockSpec((1, 1, 1), lambda b, t: (b, 0, 0)),
        ],
        out_shape=[
            jax.ShapeDtypeStruct((_B, 1, _N), jnp.float32),
            jax.ShapeDtypeStruct((_B, 1, 1), jnp.float32),
        ],
    )(features, W1, mean, var, gamma, beta, W2, b2)

    keys = pl.pallas_call(
        _keys_body,
        grid=(_B, _NT),
        in_specs=[
            pl.BlockSpec((1, 1, _TILE), lambda b, t: (b, 0, t)),
            pl.BlockSpec((1, 1, 1), lambda b, t: (b, 0, 0)),
            pl.BlockSpec((1, 1, _TILE), lambda b, t: (b, 0, t)),
        ],
        out_specs=pl.BlockSpec((1, 1, _TILE), lambda b, t: (b, 0, t)),
        out_shape=jax.ShapeDtypeStruct((_B, 1, _N), jnp.float32),
    )(act, z, gumbel)
    return keys[:, 0, :]


_NSUB = 16
_WPB = 8
_SPW = _S // _WPB
_SH = _N // _WPB
_NCK = _SH // 16
_PAD0 = _S
_SOUT = _S + 16


def _monotone_i32(kf):
    b = lax.bitcast_convert_type(kf, jnp.int32)
    return b ^ ((b >> 31) & jnp.int32(0x7FFFFFFF))


def _sc_topk_body(keys_hbm, idx_hbm,
                  keysb, mu, smu_loc, sidx_loc, cnt8, c88,
                  allmu, allidx, ranks_loc, vals_loc, idxg_v,
                  cnts_spm, smu_spm, sidx_spm, sout_spm):
    c = lax.axis_index("c")
    s = lax.axis_index("s")
    bl = s // _WPB
    b = c * 2 + bl
    g8 = s % _WPB

    ones = jnp.ones((16,), jnp.int32)
    zeros = jnp.zeros((16,), jnp.int32)
    iota16 = lax.iota(jnp.int32, 16)

    pltpu.sync_copy(keys_hbm.at[b, pl.ds(g8 * _SH, _SH)], keysb)

    @pl.loop(0, _NCK)
    def _(ch):
        mu[pl.ds(ch * 16, 16)] = _monotone_i32(keysb[pl.ds(ch * 16, 16)])

    def _count_ge(thr):
        def body(ch, acc):
            base = pl.multiple_of(ch * 64, 64)
            for u in range(4):
                m = mu[pl.ds(base + u * 16, 16)] >= thr
                acc = acc + jnp.where(m, ones, zeros)
            return acc
        acc = lax.fori_loop(0, _NCK // 4, body, zeros)
        return jnp.sum(acc)

    def _publish_and_sum(val, slot):
        cnt8[...] = jnp.full((16,), val, jnp.int32)
        pltpu.sync_copy(
            cnt8, cnts_spm.at[pl.ds(slot * 512 + bl * 256 + g8 * 32, 16)])
        plsc.subcore_barrier()
        pltpu.sync_copy(cnts_spm.at[pl.ds(slot * 512 + bl * 256, 256)], c88)
        tot = jnp.int32(0)
        pre = jnp.int32(0)
        for j in range(_WPB):
            vj = c88[pl.ds(32 * j, 16)][0]
            tot = tot + vj
            pre = pre + jnp.where(jnp.int32(j) < g8, vj, 0)
        return tot, pre

    tot0, _ = _publish_and_sum(_count_ge(jnp.int32(0)), 0)
    T = jnp.where(tot0 >= _S, jnp.int32(0), jnp.int32(-2**31))
    for bit in range(30, 9, -1):
        cand = T | jnp.int32(1 << bit)
        tot, _ = _publish_and_sum(_count_ge(cand), (31 - bit) % 2)
        T = jnp.where(tot >= _S, cand, T)

    def cbody(ch, cnt):
        kv = mu[pl.ds(pl.multiple_of(ch * 16, 16), 16)]
        m = kv >= T
        mi = jnp.where(m, ones, zeros)
        pos = cnt + jnp.cumsum(mi) - 1
        plsc.store_scatter(smu_loc, [pos], kv, mask=m)
        gi = jnp.full((16,), g8 * _SH + ch * 16, jnp.int32) + iota16
        plsc.store_scatter(sidx_loc, [pos], gi, mask=m)
        return cnt + jnp.sum(mi)

    cntw = lax.fori_loop(0, _NCK, cbody, jnp.int32(0))
    padded = (cntw + 15) & ~jnp.int32(15)
    padm = iota16 < (padded - cntw)
    ppos = cntw + iota16
    plsc.store_scatter(smu_loc, [ppos],
                       jnp.full((16,), -2**31, jnp.int32), mask=padm)
    plsc.store_scatter(sidx_loc, [ppos], jnp.zeros((16,), jnp.int32),
                       mask=padm)

    totp, base = _publish_and_sum(padded, 0)

    off = jnp.int32(0)
    for sz in (2048, 1024, 512, 256, 128, 64, 32, 16):
        cond = (padded & sz) != 0

        @pl.when(cond)
        def _():
            o1 = pl.multiple_of(off, 16)
            o2 = pl.multiple_of(bl * _N + base + off, 16)
            pltpu.sync_copy(smu_loc.at[pl.ds(o1, sz)],
                            smu_spm.at[pl.ds(o2, sz)])
            pltpu.sync_copy(sidx_loc.at[pl.ds(o1, sz)],
                            sidx_spm.at[pl.ds(o2, sz)])
        off = off + jnp.where(cond, sz, 0)
    plsc.subcore_barrier()

    off = jnp.int32(0)
    for sz in (16384, 8192, 4096, 2048, 1024, 512, 256, 128, 64, 32, 16):
        cond = (totp & sz) != 0

        @pl.when(cond)
        def _():
            o1 = pl.multiple_of(off, 16)
            o2 = pl.multiple_of(bl * _N + off, 16)
            pltpu.sync_copy(smu_spm.at[pl.ds(o2, sz)],
                            allmu.at[pl.ds(o1, sz)])
            pltpu.sync_copy(sidx_spm.at[pl.ds(o2, sz)],
                            allidx.at[pl.ds(o1, sz)])
        off = off + jnp.where(cond, sz, 0)

    @pl.loop(0, _SOUT // 16 + 1)
    def _(r):
        row = pl.multiple_of(2 * r, 2)
        ranks_loc[row] = jnp.full((16,), bl * _SOUT + _PAD0, jnp.int32) \
            + (iota16 & 7)

    nch = totp // 16

    def rbody(j, jout):
        jv = jnp.full((16,), j, jnp.int32)
        mu_jv = plsc.load_gather(smu_loc, [jv])
        idx_jv = plsc.load_gather(sidx_loc, [jv])

        def rin(ch, acc):
            cs = pl.multiple_of(ch * 16, 16)
            cmu = allmu[pl.ds(cs, 16)]
            cidx = allidx[pl.ds(cs, 16)]
            hit = (cmu > mu_jv) | ((cmu == mu_jv) & (cidx < idx_jv))
            return acc + jnp.where(hit, ones, zeros)

        rank = jnp.sum(lax.fori_loop(0, nch, rin, zeros))

        lane0 = iota16 == 0

        @pl.when(rank < _S)
        def _():
            r0 = jnp.full((16,), 2 * (jout // 16), jnp.int32)
            r1 = jnp.full((16,), jout % 16, jnp.int32)
            plsc.store_scatter(ranks_loc, [r0, r1],
                               jnp.full((16,), bl * _SOUT + rank, jnp.int32),
                               mask=lane0)
            plsc.store_scatter(vals_loc, [r0, r1], idx_jv, mask=lane0)

        return jout + jnp.where(rank < _S, 1, 0)

    jout = lax.fori_loop(0, cntw, rbody, jnp.int32(0))

    nout = (jout + 15) // 16

    def sbody(ch, _):
        row = pl.multiple_of(2 * ch, 2)
        pltpu.sync_copy(vals_loc.at[row],
                        sout_spm.at[ranks_loc.at[row]])
        return 0

    lax.fori_loop(0, nout, sbody, 0)
    plsc.subcore_barrier()

    pltpu.sync_copy(sout_spm.at[pl.ds(bl * _SOUT + g8 * _SPW, _SPW)], idxg_v)
    pltpu.sync_copy(idxg_v, idx_hbm.at[b, pl.ds(g8 * _SPW, _SPW)])


def _sc_gather_body(idx_hbm, tab_hbm, orows_hbm, idxg_v, rows_v, sem):
    c = lax.axis_index("c")
    s = lax.axis_index("s")
    w = c * _NSUB + s
    b = w // _WPB
    g8 = w % _WPB
    pltpu.sync_copy(idx_hbm.at[b, pl.ds(g8 * _SPW, _SPW)], idxg_v)
    pltpu.async_copy(tab_hbm.at[b].at[idxg_v], rows_v, sem).wait()
    pltpu.sync_copy(rows_v, orows_hbm.at[b, pl.ds(g8 * _SPW, _SPW)])


def _sc_compiler_params():
    cp = pltpu.CompilerParams()
    if "needs_layout_passes" in pltpu.CompilerParams.__dataclass_fields__:
        cp = dataclasses.replace(cp, needs_layout_passes=False)
    return cp


def _sc_topk(keys):
    mesh = plsc.VectorSubcoreMesh(core_axis_name="c", subcore_axis_name="s")
    kfn = pl.kernel(
        _sc_topk_body,
        mesh=mesh,
        compiler_params=_sc_compiler_params(),
        out_type=jax.ShapeDtypeStruct((_B, _S), jnp.int32),
        scratch_types=[
            pltpu.VMEM((_SH,), jnp.float32),
            pltpu.VMEM((_SH,), jnp.int32),
            pltpu.VMEM((_SH + 16,), jnp.int32),
            pltpu.VMEM((_SH + 16,), jnp.int32),
            pltpu.VMEM((16,), jnp.int32),
            pltpu.VMEM((256,), jnp.int32),
            pltpu.VMEM((_N,), jnp.int32),
            pltpu.VMEM((_N,), jnp.int32),
            pltpu.VMEM((2 * (_SOUT // 16 + 1), 16), jnp.int32),
            pltpu.VMEM((2 * (_SOUT // 16 + 1), 16), jnp.int32),
            pltpu.VMEM((_SPW,), jnp.int32),
            pltpu.VMEM_SHARED((1024,), jnp.int32),
            pltpu.VMEM_SHARED((2 * _N,), jnp.int32),
            pltpu.VMEM_SHARED((2 * _N,), jnp.int32),
            pltpu.VMEM_SHARED((2 * _SOUT,), jnp.int32),
        ],
    )
    return kfn(keys)


def _sc_gather(idx, table):
    mesh = plsc.VectorSubcoreMesh(core_axis_name="c", subcore_axis_name="s")
    kfn = pl.kernel(
        _sc_gather_body,
        mesh=mesh,
        compiler_params=_sc_compiler_params(),
        out_type=jax.ShapeDtypeStruct((_B, _S, _TW), jnp.float32),
        scratch_types=[
            pltpu.VMEM((_SPW,), jnp.int32),
            pltpu.VMEM((_SPW, _TW), jnp.float32),
            pltpu.SemaphoreType.DMA,
        ],
    )
    return kfn(idx, table)


def _untangle_body(rows_ref, oxyz_ref, ofea_ref):
    g = rows_ref[0]
    ofea_ref[0] = jnp.transpose(g[:, 0:_C])
    oxyz_ref[0] = g[:, _C:_C + 3]


def _untangle(rows):
    return pl.pallas_call(
        _untangle_body,
        grid=(_B,),
        in_specs=[pl.BlockSpec((1, _S, _TW), lambda b: (b, 0, 0))],
        out_specs=[
            pl.BlockSpec((1, _S, 3), lambda b: (b, 0, 0)),
            pl.BlockSpec((1, _C, _S), lambda b: (b, 0, 0)),
        ],
        out_shape=[
            jax.ShapeDtypeStruct((_B, _S, 3), jnp.float32),
            jax.ShapeDtypeStruct((_B, _C, _S), jnp.float32),
        ],
    )(rows)


def kernel(points_xyz, features, W1, gamma, beta, running_mean, running_var,
           W2, b2):
    u = jax.random.uniform(jax.random.key(42), (_B, _N),
                           minval=1e-10, maxval=1.0)
    gumbel = -jnp.log(-jnp.log(u))

    keys = _compute_keys(
        features, W1, running_mean[:, None], running_var[:, None],
        gamma[:, None], beta[:, None], W2, b2[:, None], gumbel[:, None, :])
    table = _build_table(points_xyz, features)
    idx = _sc_topk(keys)
    rows = _sc_gather(idx, table)
    new_xyz, new_fea = _untangle(rows)
    return new_xyz, new_fea, idx
